# baseline (speedup 1.0000x reference)
import jax
import jax.numpy as jnp
from jax import lax
from jax.experimental import pallas as pl
from jax.experimental.pallas import tpu as pltpu

_N = 8192
_D = 256
_BLK = 2048
_G = _N // _BLK


def _body(x_ref, w_ref, out_ref, dist_sc, runmin_s, runidx_s):
    i = pl.program_id(0)

    @pl.when(i == 0)
    def _init():
        runmin_s[0] = jnp.float32(jnp.inf)

    w = w_ref[...]
    xb = x_ref[...]
    d = w - xb
    dist2 = jnp.sum(d * d, axis=1, keepdims=True)
    dist_sc[pl.ds(i * _BLK, _BLK), :] = dist2
    m = jnp.min(dist2)
    runmin_s[0] = jnp.minimum(m, runmin_s[0])

    @pl.when(i == _G - 1)
    def _emit():
        mglob = runmin_s[0]
        ds_all = dist_sc[...]
        rowid = lax.broadcasted_iota(jnp.int32, (_N, 1), 0)
        cand = jnp.where(ds_all == mglob, rowid, jnp.int32(_N))
        win = jnp.min(cand)
        rows = lax.broadcasted_iota(jnp.int32, (_N // 128, 128), 0)
        cols = lax.broadcasted_iota(jnp.int32, (_N // 128, 128), 1)
        out_ref[...] = ((rows * 128 + cols) == win).astype(jnp.float32)


_distance_argmin = pl.pallas_call(
    _body,
    grid=(_G,),
    in_specs=[
        pl.BlockSpec((1, _D), lambda i: (0, 0)),
        pl.BlockSpec((_BLK, _D), lambda i: (i, 0)),
    ],
    out_specs=pl.BlockSpec((_N // 128, 128), lambda i: (0, 0)),
    out_shape=jax.ShapeDtypeStruct((_N // 128, 128), jnp.float32),
    scratch_shapes=[
        pltpu.VMEM((_N, 1), jnp.float32),
        pltpu.SMEM((1,), jnp.float32),
        pltpu.SMEM((1,), jnp.int32),
    ],
)


def kernel(x, reward, weights, thresholds):
    act = _distance_argmin(x.reshape(1, _D), weights)
    return act.reshape(_N)


# speedup vs baseline: 1.1056x; 1.1056x over previous
import jax
import jax.numpy as jnp
from jax import lax
from jax.experimental import pallas as pl
from jax.experimental.pallas import tpu as pltpu

_N = 8192
_D = 256
_BLK = 4096
_G = _N // _BLK


def _body(x_ref, w_ref, out_ref, dist_sc, runmin_s, runidx_s):
    i = pl.program_id(0)

    @pl.when(i == 0)
    def _init():
        runmin_s[0] = jnp.float32(jnp.inf)

    w = w_ref[...]
    xb = x_ref[...]
    d = w - xb
    dist2 = jnp.sum(d * d, axis=1, keepdims=True)
    dist_sc[pl.ds(i * _BLK, _BLK), :] = dist2
    m = jnp.min(dist2)
    runmin_s[0] = jnp.minimum(m, runmin_s[0])

    @pl.when(i == _G - 1)
    def _emit():
        mglob = runmin_s[0]
        ds_all = dist_sc[...]
        rowid = lax.broadcasted_iota(jnp.int32, (_N, 1), 0)
        cand = jnp.where(ds_all == mglob, rowid, jnp.int32(_N))
        win = jnp.min(cand)
        rows = lax.broadcasted_iota(jnp.int32, (_N // 128, 128), 0)
        cols = lax.broadcasted_iota(jnp.int32, (_N // 128, 128), 1)
        out_ref[...] = ((rows * 128 + cols) == win).astype(jnp.float32)


_distance_argmin = pl.pallas_call(
    _body,
    grid=(_G,),
    in_specs=[
        pl.BlockSpec((1, _D), lambda i: (0, 0)),
        pl.BlockSpec((_BLK, _D), lambda i: (i, 0)),
    ],
    out_specs=pl.BlockSpec((_N // 128, 128), lambda i: (0, 0)),
    out_shape=jax.ShapeDtypeStruct((_N // 128, 128), jnp.float32),
    scratch_shapes=[
        pltpu.VMEM((_N, 1), jnp.float32),
        pltpu.SMEM((1,), jnp.float32),
        pltpu.SMEM((1,), jnp.int32),
    ],
)


def kernel(x, reward, weights, thresholds):
    act = _distance_argmin(x.reshape(1, _D), weights)
    return act.reshape(_N)


# speedup vs baseline: 1.2979x; 1.1739x over previous
import jax
import jax.numpy as jnp
from jax import lax
from jax.experimental import pallas as pl
from jax.experimental.pallas import tpu as pltpu

_N = 8192
_D = 256
_BLK = 4096
_G = _N // _BLK


def _body(x_ref, w_ref, out_ref, runmin_s, runidx_s):
    i = pl.program_id(0)

    @pl.when(i == 0)
    def _init():
        runmin_s[0] = jnp.float32(jnp.inf)
        runidx_s[0] = jnp.int32(0)

    w = w_ref[...]
    xb = x_ref[...]
    d = w - xb
    dist2 = jnp.sum(d * d, axis=1, keepdims=True)
    m = jnp.min(dist2)
    rowid = lax.broadcasted_iota(jnp.int32, (_BLK, 1), 0) + i * _BLK
    cand = jnp.where(dist2 == m, rowid, jnp.int32(_N))
    li = jnp.min(cand)

    better = m < runmin_s[0]
    runmin_s[0] = jnp.where(better, m, runmin_s[0])
    runidx_s[0] = jnp.where(better, li, runidx_s[0])

    @pl.when(i == _G - 1)
    def _emit():
        win = runidx_s[0]
        rows = lax.broadcasted_iota(jnp.int32, (_N // 128, 128), 0)
        cols = lax.broadcasted_iota(jnp.int32, (_N // 128, 128), 1)
        out_ref[...] = ((rows * 128 + cols) == win).astype(jnp.float32)


_distance_argmin = pl.pallas_call(
    _body,
    grid=(_G,),
    in_specs=[
        pl.BlockSpec((1, _D), lambda i: (0, 0)),
        pl.BlockSpec((_BLK, _D), lambda i: (i, 0)),
    ],
    out_specs=pl.BlockSpec((_N // 128, 128), lambda i: (0, 0)),
    out_shape=jax.ShapeDtypeStruct((_N // 128, 128), jnp.float32),
    scratch_shapes=[
        pltpu.SMEM((1,), jnp.float32),
        pltpu.SMEM((1,), jnp.int32),
    ],
)


def kernel(x, reward, weights, thresholds):
    act = _distance_argmin(x.reshape(1, _D), weights)
    return act.reshape(_N)
